# baseline (speedup 1.0000x reference)
import functools

import jax
import jax.numpy as jnp
from jax import lax
from jax.experimental import pallas as pl
from jax.experimental.pallas import tpu as pltpu
from jax.experimental.pallas import tpu_sc as plsc

_B, _L, _A = 4096, 2048, 20
_AP = 21
_NC, _NS, _LANES = 2, 16, 16
_NW = _NC * _NS

_B_SC = 2048
_B_TC = _B - _B_SC
_BB = 256

_ROWS_PER_W = _B_SC // _NW
_GROUP = _LANES
_NGROUPS = _ROWS_PER_W // _GROUP
_CHUNKS = _L // _LANES


def _logsumexp_total_body(logits_ref, out_ref):
    lg = logits_ref[...]
    m = jnp.max(lg, axis=-1, keepdims=True)
    s = jnp.sum(jnp.exp(lg - m), axis=-1, keepdims=True)
    lse = m + jnp.log(s)
    out_ref[...] = jnp.sum(lse).reshape(1, 1)


_logsumexp_total = pl.pallas_call(
    _logsumexp_total_body,
    out_shape=jax.ShapeDtypeStruct((1, 1), jnp.float32),
)


def _tc_gather_body(x_ref, tbl_ref, out_ref):
    xb = x_ref[...]
    cols = [tbl_ref[a, :].reshape(1, _L) for a in range(_A)]

    m0 = (xb & 1) != 0
    m1 = (xb & 2) != 0
    m2 = (xb & 4) != 0
    m3 = (xb & 8) != 0
    m4 = (xb & 16) != 0

    u = [jnp.where(m0, cols[2 * i + 1], cols[2 * i]) for i in range(10)]
    v = [jnp.where(m1, u[2 * i + 1], u[2 * i]) for i in range(5)]
    w0 = jnp.where(m2, v[1], v[0])
    w1 = jnp.where(m2, v[3], v[2])
    z0 = jnp.where(m3, w1, w0)
    g = jnp.where(m4, v[4], z0)
    out_ref[...] = jnp.sum(g, axis=1, keepdims=True)


_tc_gather_sum = pl.pallas_call(
    _tc_gather_body,
    grid=(_B_TC // _BB,),
    in_specs=[
        pl.BlockSpec((_BB, _L), lambda i: (i, 0)),
        pl.BlockSpec((_A, _L), lambda i: (0, 0)),
    ],
    out_specs=pl.BlockSpec((_BB, 1), lambda i: (i, 0)),
    out_shape=jax.ShapeDtypeStruct((_B_TC, 1), jnp.float32),
)


def _gather_sum_body(x_hbm, table_hbm, out_hbm, table_v, xbuf0, xbuf1, out_v,
                     sem0, sem1, tsem):
    cid = lax.axis_index("c")
    sid = lax.axis_index("s")
    wid = sid * _NC + cid
    row0 = wid * _ROWS_PER_W

    tcopy = pltpu.async_copy(table_hbm, table_v, tsem)

    sems = [sem0, sem1]
    bufs = [xbuf0, xbuf1]

    def start_copy(g):
        return pltpu.async_copy(
            x_hbm.at[pl.ds(_B_TC + row0 + g * _GROUP, _GROUP), :],
            bufs[g % 2], sems[g % 2])

    pending = start_copy(0)
    lane = lax.iota(jnp.int32, _LANES)
    lane_off = lane * _AP
    zero = jnp.zeros((_LANES,), jnp.float32)
    tcopy.wait()

    for g in range(_NGROUPS):
        cur = pending
        if g + 1 < _NGROUPS:
            pending = start_copy(g + 1)
        cur.wait()
        xb = bufs[g % 2]

        def row_body(r, resvec, xb=xb):

            @plsc.parallel_loop(0, _CHUNKS, step=2, unroll=4,
                                carry=(zero, zero))
            def acc(j, carry):
                a0, a1 = carry
                x0 = xb[r, pl.ds(j * _LANES, _LANES)]
                x1 = xb[r, pl.ds((j + 1) * _LANES, _LANES)]
                t0 = plsc.load_gather(
                    table_v, [x0 + lane_off + j * (_LANES * _AP)])
                t1 = plsc.load_gather(
                    table_v, [x1 + lane_off + (j + 1) * (_LANES * _AP)])
                return (a0 + t0, a1 + t1)

            rowsum = jnp.sum(acc[0] + acc[1])
            return jnp.where(lane == r, rowsum, resvec)

        out_v[pl.ds(g * _GROUP, _GROUP)] = lax.fori_loop(
            0, _GROUP, row_body, zero)

    pltpu.sync_copy(out_v, out_hbm.at[pl.ds(row0, _ROWS_PER_W)])


_gather_sum = pl.kernel(
    _gather_sum_body,
    out_type=jax.ShapeDtypeStruct((_B_SC,), jnp.float32),
    mesh=plsc.VectorSubcoreMesh(core_axis_name="c", subcore_axis_name="s"),
    compiler_params=pltpu.CompilerParams(needs_layout_passes=False),
    scratch_types=[
        pltpu.VMEM((_L * _AP,), jnp.float32),
        pltpu.VMEM((_GROUP, _L), jnp.int32),
        pltpu.VMEM((_GROUP, _L), jnp.int32),
        pltpu.VMEM((_ROWS_PER_W,), jnp.float32),
        pltpu.SemaphoreType.DMA,
        pltpu.SemaphoreType.DMA,
        pltpu.SemaphoreType.DMA,
    ],
)


def kernel(x, logits):
    table = jnp.pad(logits, ((0, 0), (0, _AP - _A))).reshape(-1)
    raw_sc = _gather_sum(x, table)
    raw_tc = _tc_gather_sum(x[:_B_TC], logits.T)
    c = _logsumexp_total(logits)
    return jnp.concatenate([raw_tc[:, 0], raw_sc]) - c[0, 0]


# speedup vs baseline: 1.3046x; 1.3046x over previous
import functools

import jax
import jax.numpy as jnp
from jax import lax
from jax.experimental import pallas as pl
from jax.experimental.pallas import tpu as pltpu
from jax.experimental.pallas import tpu_sc as plsc

_B, _L, _A = 4096, 2048, 20
_AP = 21
_SKEW = 17
_NC, _NS, _LANES = 2, 16, 16
_NW = _NC * _NS

_B_SC = 2048
_B_TC = _B - _B_SC
_BB = 256

_ROWS_PER_W = _B_SC // _NW
_GROUP = _LANES
_NGROUPS = _ROWS_PER_W // _GROUP
_CHUNKS = _L // _LANES


def _logsumexp_total_body(logits_ref, out_ref):
    lg = logits_ref[...]
    m = jnp.max(lg, axis=-1, keepdims=True)
    s = jnp.sum(jnp.exp(lg - m), axis=-1, keepdims=True)
    lse = m + jnp.log(s)
    out_ref[...] = jnp.sum(lse).reshape(1, 1)


_logsumexp_total = pl.pallas_call(
    _logsumexp_total_body,
    out_shape=jax.ShapeDtypeStruct((1, 1), jnp.float32),
)


def _tc_gather_body(x_ref, tbl_ref, out_ref):
    xb = x_ref[...]
    cols = [tbl_ref[a, :].reshape(1, _L) for a in range(_A)]

    m0 = (xb & 1) != 0
    m1 = (xb & 2) != 0
    m2 = (xb & 4) != 0
    m3 = (xb & 8) != 0
    m4 = (xb & 16) != 0

    u = [jnp.where(m0, cols[2 * i + 1], cols[2 * i]) for i in range(10)]
    v = [jnp.where(m1, u[2 * i + 1], u[2 * i]) for i in range(5)]
    w0 = jnp.where(m2, v[1], v[0])
    w1 = jnp.where(m2, v[3], v[2])
    z0 = jnp.where(m3, w1, w0)
    g = jnp.where(m4, v[4], z0)
    out_ref[...] = jnp.sum(g, axis=1, keepdims=True)


_tc_gather_sum = pl.pallas_call(
    _tc_gather_body,
    grid=(_B_TC // _BB,),
    in_specs=[
        pl.BlockSpec((_BB, _L), lambda i: (i, 0)),
        pl.BlockSpec((_A, _L), lambda i: (0, 0)),
    ],
    out_specs=pl.BlockSpec((_BB, 1), lambda i: (i, 0)),
    out_shape=jax.ShapeDtypeStruct((_B_TC, 1), jnp.float32),
)


def _gather_sum_body(x_hbm, table_hbm, out_hbm, table_v, xbuf0, xbuf1, out_v,
                     tr_v, sem0, sem1, tsem):
    cid = lax.axis_index("c")
    sid = lax.axis_index("s")
    wid = sid * _NC + cid
    row0 = wid * _ROWS_PER_W

    tcopy = pltpu.async_copy(table_hbm, table_v, tsem)

    sems = [sem0, sem1]
    bufs = [xbuf0, xbuf1]

    def start_copy(g):
        return pltpu.async_copy(
            x_hbm.at[pl.ds(_B_TC + row0 + g * _GROUP, _GROUP), :],
            bufs[g % 2], sems[g % 2])

    pending = start_copy(0)
    lane = lax.iota(jnp.int32, _LANES)
    lane_off = lane * _AP
    zero = jnp.zeros((_LANES,), jnp.float32)
    tcopy.wait()

    for g in range(_NGROUPS):
        cur = pending
        if g + 1 < _NGROUPS:
            pending = start_copy(g + 1)
        cur.wait()
        xb = bufs[g % 2]

        @plsc.parallel_loop(0, _CHUNKS, carry=(zero,) * _GROUP)
        def accs(j, carry, xb=xb):
            jbase = j * _LANES
            out = []
            for r in range(_GROUP):
                xv = xb[r, pl.ds(jbase, _LANES)]
                t = plsc.load_gather(
                    table_v, [xv + lane_off + jbase * _AP])
                out.append(carry[r] + t)
            return tuple(out)

        for r in range(_GROUP):
            plsc.store_scatter(tr_v, [lane + r * _SKEW], accs[r])
        res = zero
        for i in range(_LANES):
            res = res + plsc.load_gather(tr_v, [lane * _SKEW + i])
        out_v[pl.ds(g * _GROUP, _GROUP)] = res

    pltpu.sync_copy(out_v, out_hbm.at[pl.ds(row0, _ROWS_PER_W)])


_gather_sum = pl.kernel(
    _gather_sum_body,
    out_type=jax.ShapeDtypeStruct((_B_SC,), jnp.float32),
    mesh=plsc.VectorSubcoreMesh(core_axis_name="c", subcore_axis_name="s"),
    compiler_params=pltpu.CompilerParams(needs_layout_passes=False),
    scratch_types=[
        pltpu.VMEM((_L * _AP,), jnp.float32),
        pltpu.VMEM((_GROUP, _L), jnp.int32),
        pltpu.VMEM((_GROUP, _L), jnp.int32),
        pltpu.VMEM((_ROWS_PER_W,), jnp.float32),
        pltpu.VMEM((_GROUP * _SKEW,), jnp.float32),
        pltpu.SemaphoreType.DMA,
        pltpu.SemaphoreType.DMA,
        pltpu.SemaphoreType.DMA,
    ],
)


def kernel(x, logits):
    table = jnp.pad(logits, ((0, 0), (0, _AP - _A))).reshape(-1)
    raw_sc = _gather_sum(x, table)
    raw_tc = _tc_gather_sum(x, logits.T)
    c = _logsumexp_total(logits)
    return jnp.concatenate([raw_tc[:, 0], raw_sc]) - c[0, 0]


# speedup vs baseline: 1.4845x; 1.1379x over previous
import functools

import jax
import jax.numpy as jnp
from jax import lax
from jax.experimental import pallas as pl
from jax.experimental.pallas import tpu as pltpu
from jax.experimental.pallas import tpu_sc as plsc

_B, _L, _A = 4096, 2048, 20
_AP = 21
_SKEW = 17
_NC, _NS, _LANES = 2, 16, 16
_NW = _NC * _NS

_B_SC = 3072
_B_TC = _B - _B_SC
_BB = 256

_ROWS_PER_W = _B_SC // _NW
_GROUP = _LANES
_NGROUPS = _ROWS_PER_W // _GROUP
_CHUNKS = _L // _LANES


def _tc_gather_body(x_ref, tbl_ref, out_ref, c_ref):
    xb = x_ref[...]
    cols = [tbl_ref[a, :].reshape(1, _L) for a in range(_A)]

    m0 = (xb & 1) != 0
    m1 = (xb & 2) != 0
    m2 = (xb & 4) != 0
    m3 = (xb & 8) != 0
    m4 = (xb & 16) != 0

    u = [jnp.where(m0, cols[2 * i + 1], cols[2 * i]) for i in range(10)]
    v = [jnp.where(m1, u[2 * i + 1], u[2 * i]) for i in range(5)]
    w0 = jnp.where(m2, v[1], v[0])
    w1 = jnp.where(m2, v[3], v[2])
    z0 = jnp.where(m3, w1, w0)
    g = jnp.where(m4, v[4], z0)
    out_ref[...] = jnp.sum(g, axis=1, keepdims=True)

    @pl.when(pl.program_id(0) == 0)
    def _():
        tb = tbl_ref[...]
        m = jnp.max(tb, axis=0, keepdims=True)
        s = jnp.sum(jnp.exp(tb - m), axis=0, keepdims=True)
        c_ref[...] = jnp.sum(m + jnp.log(s)).reshape(1, 1)


_tc_gather_sum = pl.pallas_call(
    _tc_gather_body,
    grid=(_B_TC // _BB,),
    in_specs=[
        pl.BlockSpec((_BB, _L), lambda i: (i, 0)),
        pl.BlockSpec((_A, _L), lambda i: (0, 0)),
    ],
    out_specs=[
        pl.BlockSpec((_BB, 1), lambda i: (i, 0)),
        pl.BlockSpec((1, 1), lambda i: (0, 0)),
    ],
    out_shape=[
        jax.ShapeDtypeStruct((_B_TC, 1), jnp.float32),
        jax.ShapeDtypeStruct((1, 1), jnp.float32),
    ],
)


def _gather_sum_body(x_hbm, table_hbm, out_hbm, table_v, xbuf0, xbuf1, out_v,
                     tr_v, sem0, sem1, tsem):
    cid = lax.axis_index("c")
    sid = lax.axis_index("s")
    wid = sid * _NC + cid
    row0 = wid * _ROWS_PER_W

    tcopy = pltpu.async_copy(table_hbm, table_v, tsem)

    sems = [sem0, sem1]
    bufs = [xbuf0, xbuf1]

    def start_copy(g):
        return pltpu.async_copy(
            x_hbm.at[pl.ds(_B_TC + row0 + g * _GROUP, _GROUP), :],
            bufs[g % 2], sems[g % 2])

    pending = start_copy(0)
    lane = lax.iota(jnp.int32, _LANES)
    lane_off = lane * _AP
    zero = jnp.zeros((_LANES,), jnp.float32)
    tcopy.wait()

    for g in range(_NGROUPS):
        cur = pending
        if g + 1 < _NGROUPS:
            pending = start_copy(g + 1)
        cur.wait()
        xb = bufs[g % 2]

        @plsc.parallel_loop(0, _CHUNKS, carry=(zero,) * _GROUP)
        def accs(j, carry, xb=xb):
            jbase = j * _LANES
            out = []
            for r in range(_GROUP):
                xv = xb[r, pl.ds(jbase, _LANES)]
                t = plsc.load_gather(
                    table_v, [xv + lane_off + jbase * _AP])
                out.append(carry[r] + t)
            return tuple(out)

        for r in range(_GROUP):
            plsc.store_scatter(tr_v, [lane + r * _SKEW], accs[r])
        res = zero
        for i in range(_LANES):
            res = res + plsc.load_gather(tr_v, [lane * _SKEW + i])
        out_v[pl.ds(g * _GROUP, _GROUP)] = res

    pltpu.sync_copy(out_v, out_hbm.at[pl.ds(row0, _ROWS_PER_W)])


_gather_sum = pl.kernel(
    _gather_sum_body,
    out_type=jax.ShapeDtypeStruct((_B_SC,), jnp.float32),
    mesh=plsc.VectorSubcoreMesh(core_axis_name="c", subcore_axis_name="s"),
    compiler_params=pltpu.CompilerParams(needs_layout_passes=False),
    scratch_types=[
        pltpu.VMEM((_L * _AP,), jnp.float32),
        pltpu.VMEM((_GROUP, _L), jnp.int32),
        pltpu.VMEM((_GROUP, _L), jnp.int32),
        pltpu.VMEM((_ROWS_PER_W,), jnp.float32),
        pltpu.VMEM((_GROUP * _SKEW,), jnp.float32),
        pltpu.SemaphoreType.DMA,
        pltpu.SemaphoreType.DMA,
        pltpu.SemaphoreType.DMA,
    ],
)


def kernel(x, logits):
    table = jnp.pad(logits, ((0, 0), (0, _AP - _A))).reshape(-1)
    raw_sc = _gather_sum(x, table)
    raw_tc, c = _tc_gather_sum(x, logits.T)
    return jnp.concatenate([raw_tc[:, 0], raw_sc]) - c[0, 0]


# speedup vs baseline: 1.6360x; 1.1021x over previous
import functools

import jax
import jax.numpy as jnp
from jax import lax
from jax.experimental import pallas as pl
from jax.experimental.pallas import tpu as pltpu
from jax.experimental.pallas import tpu_sc as plsc

_B, _L, _A = 4096, 2048, 20
_AP = 21
_SKEW = 17
_NC, _NS, _LANES = 2, 16, 16
_NW = _NC * _NS

_B_SC = 2560
_B_TC = _B - _B_SC
_BB = 256

_ROWS_PER_W = _B_SC // _NW
_GROUP = _LANES
_NGROUPS = _ROWS_PER_W // _GROUP
_CHUNKS = _L // _LANES


def _tc_gather_body(x_ref, tbl_ref, out_ref, c_ref):
    xb = x_ref[...]
    cols = [tbl_ref[a, :].reshape(1, _L) for a in range(_A)]

    m0 = (xb & 1) != 0
    m1 = (xb & 2) != 0
    m2 = (xb & 4) != 0
    m3 = (xb & 8) != 0
    m4 = (xb & 16) != 0

    u = [jnp.where(m0, cols[2 * i + 1], cols[2 * i]) for i in range(10)]
    v = [jnp.where(m1, u[2 * i + 1], u[2 * i]) for i in range(5)]
    w0 = jnp.where(m2, v[1], v[0])
    w1 = jnp.where(m2, v[3], v[2])
    z0 = jnp.where(m3, w1, w0)
    g = jnp.where(m4, v[4], z0)
    out_ref[...] = jnp.sum(g, axis=1, keepdims=True)

    @pl.when(pl.program_id(0) == 0)
    def _():
        tb = tbl_ref[...]
        m = jnp.max(tb, axis=0, keepdims=True)
        s = jnp.sum(jnp.exp(tb - m), axis=0, keepdims=True)
        c_ref[...] = jnp.sum(m + jnp.log(s)).reshape(1, 1)


_tc_gather_sum = pl.pallas_call(
    _tc_gather_body,
    grid=(_B_TC // _BB,),
    in_specs=[
        pl.BlockSpec((_BB, _L), lambda i: (i, 0)),
        pl.BlockSpec((_A, _L), lambda i: (0, 0)),
    ],
    out_specs=[
        pl.BlockSpec((_BB, 1), lambda i: (i, 0)),
        pl.BlockSpec((1, 1), lambda i: (0, 0)),
    ],
    out_shape=[
        jax.ShapeDtypeStruct((_B_TC, 1), jnp.float32),
        jax.ShapeDtypeStruct((1, 1), jnp.float32),
    ],
)


def _gather_sum_body(x_hbm, table_hbm, out_hbm, table_v, xbuf, out_v,
                     tr_v, sems, tsem):
    cid = lax.axis_index("c")
    sid = lax.axis_index("s")
    wid = sid * _NC + cid
    row0 = wid * _ROWS_PER_W

    tcopy = pltpu.async_copy(table_hbm, table_v, tsem)

    def start_copy(g, slot):
        return pltpu.async_copy(
            x_hbm.at[pl.ds(_B_TC + row0 + g * _GROUP, _GROUP), :],
            xbuf.at[slot], sems.at[slot])

    start_copy(0, 0)
    lane = lax.iota(jnp.int32, _LANES)
    lane_off = lane * _AP
    zero = jnp.zeros((_LANES,), jnp.float32)
    tcopy.wait()

    def group_body(g, _):
        slot = lax.rem(g, 2)

        @pl.when(g + 1 < _NGROUPS)
        def _():
            start_copy(g + 1, lax.rem(g + 1, 2))

        pltpu.make_async_copy(
            x_hbm.at[pl.ds(_B_TC + row0 + g * _GROUP, _GROUP), :],
            xbuf.at[slot], sems.at[slot]).wait()

        @plsc.parallel_loop(0, _CHUNKS, carry=(zero,) * _GROUP)
        def accs(j, carry):
            jbase = j * _LANES
            out = []
            for r in range(_GROUP):
                xv = xbuf[slot, r, pl.ds(jbase, _LANES)]
                t = plsc.load_gather(
                    table_v, [xv + lane_off + jbase * _AP])
                out.append(carry[r] + t)
            return tuple(out)

        for r in range(_GROUP):
            plsc.store_scatter(tr_v, [lane + r * _SKEW], accs[r])
        res = zero
        for i in range(_LANES):
            res = res + plsc.load_gather(tr_v, [lane * _SKEW + i])
        out_v[pl.ds(g * _GROUP, _GROUP)] = res
        return 0

    lax.fori_loop(0, _NGROUPS, group_body, 0)
    pltpu.sync_copy(out_v, out_hbm.at[pl.ds(row0, _ROWS_PER_W)])


_gather_sum = pl.kernel(
    _gather_sum_body,
    out_type=jax.ShapeDtypeStruct((_B_SC,), jnp.float32),
    mesh=plsc.VectorSubcoreMesh(core_axis_name="c", subcore_axis_name="s"),
    compiler_params=pltpu.CompilerParams(needs_layout_passes=False),
    scratch_types=[
        pltpu.VMEM((_L * _AP,), jnp.float32),
        pltpu.VMEM((2, _GROUP, _L), jnp.int32),
        pltpu.VMEM((_ROWS_PER_W,), jnp.float32),
        pltpu.VMEM((_GROUP * _SKEW,), jnp.float32),
        pltpu.SemaphoreType.DMA((2,)),
        pltpu.SemaphoreType.DMA,
    ],
)


def kernel(x, logits):
    table = jnp.pad(logits, ((0, 0), (0, _AP - _A))).reshape(-1)
    raw_sc = _gather_sum(x, table)
    raw_tc, c = _tc_gather_sum(x, logits.T)
    return jnp.concatenate([raw_tc[:, 0], raw_sc]) - c[0, 0]


# speedup vs baseline: 1.6917x; 1.0340x over previous
import functools

import jax
import jax.numpy as jnp
from jax import lax
from jax.experimental import pallas as pl
from jax.experimental.pallas import tpu as pltpu
from jax.experimental.pallas import tpu_sc as plsc

_B, _L, _A = 4096, 2048, 20
_AP = 21
_SKEW = 17
_NC, _NS, _LANES = 2, 16, 16
_NW = _NC * _NS

_B_SC = 2560
_B_TC = _B - _B_SC
_BB = 256

_ROWS_PER_W = _B_SC // _NW
_GROUP = _LANES
_NGROUPS = _ROWS_PER_W // _GROUP
_CHUNKS = _L // _LANES


def _tc_gather_body(x_ref, tbl_ref, out_ref, c_ref):
    xb = x_ref[...]
    cols = [tbl_ref[a, :].reshape(1, _L) for a in range(_A)]

    m0 = (xb & 1) != 0
    m1 = (xb & 2) != 0
    m2 = (xb & 4) != 0
    m3 = (xb & 8) != 0
    m4 = (xb & 16) != 0

    u = [jnp.where(m0, cols[2 * i + 1], cols[2 * i]) for i in range(10)]
    v = [jnp.where(m1, u[2 * i + 1], u[2 * i]) for i in range(5)]
    w0 = jnp.where(m2, v[1], v[0])
    w1 = jnp.where(m2, v[3], v[2])
    z0 = jnp.where(m3, w1, w0)
    g = jnp.where(m4, v[4], z0)
    out_ref[...] = jnp.sum(g, axis=1, keepdims=True)

    @pl.when(pl.program_id(0) == 0)
    def _():
        tb = tbl_ref[...]
        m = jnp.max(tb, axis=0, keepdims=True)
        s = jnp.sum(jnp.exp(tb - m), axis=0, keepdims=True)
        c_ref[...] = jnp.sum(m + jnp.log(s)).reshape(1, 1)


_tc_gather_sum = pl.pallas_call(
    _tc_gather_body,
    grid=(_B_TC // _BB,),
    in_specs=[
        pl.BlockSpec((_BB, _L), lambda i: (i, 0)),
        pl.BlockSpec((_A, _L), lambda i: (0, 0)),
    ],
    out_specs=[
        pl.BlockSpec((_BB, 1), lambda i: (i, 0)),
        pl.BlockSpec((1, 1), lambda i: (0, 0)),
    ],
    out_shape=[
        jax.ShapeDtypeStruct((_B_TC, 1), jnp.float32),
        jax.ShapeDtypeStruct((1, 1), jnp.float32),
    ],
)


def _gather_sum_body(x_hbm, table_hbm, out_hbm, table_v, xbuf, out_v,
                     tr_v, sems, tsem):
    cid = lax.axis_index("c")
    sid = lax.axis_index("s")
    wid = sid * _NC + cid
    row0 = wid * _ROWS_PER_W

    tcopy = pltpu.async_copy(table_hbm, table_v, tsem)

    def start_copy(g, slot):
        return pltpu.async_copy(
            x_hbm.at[pl.ds(_B_TC + row0 + g * _GROUP, _GROUP), :],
            xbuf.at[slot], sems.at[slot])

    start_copy(0, 0)
    lane = lax.iota(jnp.int32, _LANES)
    zero = jnp.zeros((_LANES,), jnp.float32)
    tcopy.wait()

    def group_body(g, _):
        slot = lax.rem(g, 2)

        @pl.when(g + 1 < _NGROUPS)
        def _():
            start_copy(g + 1, lax.rem(g + 1, 2))

        pltpu.make_async_copy(
            x_hbm.at[pl.ds(_B_TC + row0 + g * _GROUP, _GROUP), :],
            xbuf.at[slot], sems.at[slot]).wait()

        @plsc.parallel_loop(0, _CHUNKS, carry=(zero,) * _GROUP)
        def accs(j, carry):
            lvec = lane + j * _LANES
            out = []
            for r in range(_GROUP):
                xv = xbuf[slot, r, pl.ds(j * _LANES, _LANES)]
                t = plsc.load_gather(table_v, [(xv << 11) | lvec])
                out.append(carry[r] + t)
            return tuple(out)

        for r in range(_GROUP):
            plsc.store_scatter(tr_v, [lane + r * _SKEW], accs[r])
        res = zero
        for i in range(_LANES):
            res = res + plsc.load_gather(tr_v, [lane * _SKEW + i])
        out_v[pl.ds(g * _GROUP, _GROUP)] = res
        return 0

    lax.fori_loop(0, _NGROUPS, group_body, 0)
    pltpu.sync_copy(out_v, out_hbm.at[pl.ds(row0, _ROWS_PER_W)])


_gather_sum = pl.kernel(
    _gather_sum_body,
    out_type=jax.ShapeDtypeStruct((_B_SC,), jnp.float32),
    mesh=plsc.VectorSubcoreMesh(core_axis_name="c", subcore_axis_name="s"),
    compiler_params=pltpu.CompilerParams(needs_layout_passes=False),
    scratch_types=[
        pltpu.VMEM((_A * _L,), jnp.float32),
        pltpu.VMEM((2, _GROUP, _L), jnp.int32),
        pltpu.VMEM((_ROWS_PER_W,), jnp.float32),
        pltpu.VMEM((_GROUP * _SKEW,), jnp.float32),
        pltpu.SemaphoreType.DMA((2,)),
        pltpu.SemaphoreType.DMA,
    ],
)


def kernel(x, logits):
    tbl_t = logits.T
    raw_sc = _gather_sum(x, tbl_t.reshape(-1))
    raw_tc, c = _tc_gather_sum(x, tbl_t)
    return jnp.concatenate([raw_tc[:, 0], raw_sc]) - c[0, 0]
